# baseline (speedup 1.0000x reference)
import functools

import jax
import jax.numpy as jnp
from jax import lax
from jax.experimental import pallas as pl
from jax.experimental.pallas import tpu as pltpu
from jax.experimental.pallas import tpu_sc as plsc

N_EMB = 8192
DIM = 64
BATCH = 16
SEQ = 576
TOKENS = BATCH * SEQ
TILE = 576
RB = 72
NRB = TILE // RB
NLT = N_EMB // 128

NCORES = 2
NSUB = 16
NWORKERS = NCORES * NSUB
BPW = TOKENS // NWORKERS
GCH = 96
NGR = BPW // GCH


def _argmin_body(z_ref, cb_ref, idx_ref, slab_ref, loss_ref):
    step = pl.program_id(0)
    zt = z_ref[0]
    sz = jnp.sum(zt * zt, axis=1, keepdims=True)
    zneg = zt * -2.0
    lane = lax.broadcasted_iota(jnp.int32, (1, 128), 1).astype(jnp.float32)

    total = jnp.float32(0.0)
    idx_parts = []
    for rb in range(NRB):
        r0 = rb * RB
        zr = zneg[r0:r0 + RB]
        dneg = lax.dot_general(zr, cb_ref[...], (((1,), (1,)), ((), ())),
                               preferred_element_type=jnp.float32)
        szb = jnp.broadcast_to(sz[r0:r0 + RB], (RB, 128))
        bv = szb + dneg[:, 0:128]
        bi = jnp.broadcast_to(lane, (RB, 128))
        for t in range(1, NLT):
            dt = szb + dneg[:, t * 128:(t + 1) * 128]
            ids = lane + jnp.float32(t * 128)
            take = dt < bv
            bv = jnp.where(take, dt, bv)
            bi = jnp.where(take, ids, bi)
        mv = jnp.min(bv, axis=1, keepdims=True)
        mi = jnp.min(jnp.where(bv == mv, bi, jnp.float32(N_EMB)),
                     axis=1, keepdims=True)
        idx_parts.append(mi[:, 0].astype(jnp.int32))
        total = total + jnp.sum(mv[:, 0])

    cat = jnp.concatenate(idx_parts).reshape(1, TILE)
    idx_ref[pl.ds(step % 8, 1), :] = cat
    zpad = jnp.zeros((1, 128 - GCH), jnp.int32)
    zrow = jnp.zeros((1, 128), jnp.int32)
    halves = []
    for w in range(2):
        rows = [jnp.concatenate(
                    [cat[:, (w * NGR + j) * GCH:(w * NGR + j + 1) * GCH],
                     zpad], axis=1)
                for j in range(NGR)] + [zrow] * 5
        halves.append(jnp.concatenate(rows, axis=0).reshape(1, 8, 128))
    slab_ref[...] = jnp.concatenate(halves, axis=0)

    @pl.when(step == 0)
    def _():
        loss_ref[0, 0] = 0.0

    loss_ref[0, 0] += total

    @pl.when(step == BATCH - 1)
    def _():
        m = loss_ref[0, 0] / (TOKENS * DIM)
        loss_ref[0, 0] = m + 0.25 * m


def _tc_argmin(z, codebook):
    return pl.pallas_call(
        _argmin_body,
        grid=(BATCH,),
        in_specs=[
            pl.BlockSpec((1, TILE, DIM), lambda b: (b, 0, 0)),
            pl.BlockSpec((N_EMB, DIM), lambda b: (0, 0)),
        ],
        out_specs=[
            pl.BlockSpec((8, SEQ), lambda b: (b // 8, 0)),
            pl.BlockSpec((2, 8, 128), lambda b: (b, 0, 0)),
            pl.BlockSpec(memory_space=pltpu.SMEM),
        ],
        out_shape=[
            jax.ShapeDtypeStruct((BATCH, SEQ), jnp.int32),
            jax.ShapeDtypeStruct((NWORKERS, 8, 128), jnp.int32),
            jax.ShapeDtypeStruct((1, 1), jnp.float32),
        ],
    )(z, codebook)


@functools.cache
def _make_sc_gather():
    mesh = plsc.VectorSubcoreMesh(core_axis_name="c", subcore_axis_name="s")

    @functools.partial(
        pl.kernel,
        mesh=mesh,
        out_type=jax.ShapeDtypeStruct((BATCH, SEQ, DIM), jnp.float32),
        scratch_types=[
            pltpu.VMEM((8, 128), jnp.int32),
            pltpu.VMEM((NGR * 128, DIM), jnp.float32),
            pltpu.SemaphoreType.DMA,
        ],
        compiler_params=pltpu.CompilerParams(use_tc_tiling_on_sc=False),
    )
    def _sc_gather(cb_hbm, idx_hbm, out_hbm, idx_v, rows_v, sem):
        wid = lax.axis_index("s") * NCORES + lax.axis_index("c")
        b = wid // 2
        half = wid % 2
        pltpu.sync_copy(idx_hbm.at[wid], idx_v)
        copies = [
            pltpu.async_copy(cb_hbm.at[idx_v.at[j]],
                             rows_v.at[pl.ds(j * 128, 128)], sem)
            for j in range(NGR)
        ]
        for c in copies:
            c.wait()
        for j in range(NGR):
            pltpu.sync_copy(
                rows_v.at[pl.ds(j * 128, GCH)],
                out_hbm.at[b, pl.ds(half * BPW + j * GCH, GCH)])

    return _sc_gather


def kernel(z, codebook):
    idx, slab, lsum = _tc_argmin(z, codebook)
    quantized = _make_sc_gather()(codebook, slab)
    loss = lsum[0, 0]
    return quantized, loss, idx


# speedup vs baseline: 1.4346x; 1.4346x over previous
import functools

import jax
import jax.numpy as jnp
from jax import lax
from jax.experimental import pallas as pl
from jax.experimental.pallas import tpu as pltpu
from jax.experimental.pallas import tpu_sc as plsc

N_EMB = 8192
DIM = 64
BATCH = 16
SEQ = 576
TOKENS = BATCH * SEQ
TILE = 576
RB = 72
NRB = TILE // RB
NLT = N_EMB // 128

NCORES = 2
NSUB = 16
NWORKERS = NCORES * NSUB
BPW = TOKENS // NWORKERS
GCH = 96
NGCH = BPW // GCH


def _argmin_body(z_ref, cb_ref, idx_ref, loss_ref):
    step = pl.program_id(0)
    zt = z_ref[0]
    sz = jnp.sum(zt * zt, axis=1, keepdims=True)
    zneg = zt * -2.0
    lane = lax.broadcasted_iota(jnp.int32, (1, 128), 1).astype(jnp.float32)

    total = jnp.float32(0.0)
    idx_parts = []
    for rb in range(NRB):
        r0 = rb * RB
        zr = zneg[r0:r0 + RB]
        dneg = lax.dot_general(zr, cb_ref[...], (((1,), (1,)), ((), ())),
                               preferred_element_type=jnp.float32)
        szb = jnp.broadcast_to(sz[r0:r0 + RB], (RB, 128))
        bv = szb + dneg[:, 0:128]
        bi = jnp.broadcast_to(lane, (RB, 128))
        for t in range(1, NLT):
            dt = szb + dneg[:, t * 128:(t + 1) * 128]
            ids = lane + jnp.float32(t * 128)
            take = dt < bv
            bv = jnp.where(take, dt, bv)
            bi = jnp.where(take, ids, bi)
        mv = jnp.min(bv, axis=1, keepdims=True)
        mi = jnp.min(jnp.where(bv == mv, bi, jnp.float32(N_EMB)),
                     axis=1, keepdims=True)
        idx_parts.append(mi[:, 0].astype(jnp.int32))
        total = total + jnp.sum(mv[:, 0])

    cat = jnp.concatenate(idx_parts).reshape(1, TILE)
    idx_ref[pl.ds(step % 8, 1), :] = cat

    @pl.when(step == 0)
    def _():
        loss_ref[0, 0] = 0.0

    loss_ref[0, 0] += total

    @pl.when(step == BATCH - 1)
    def _():
        m = loss_ref[0, 0] / (TOKENS * DIM)
        loss_ref[0, 0] = m + 0.25 * m


def _tc_argmin(z, codebook):
    return pl.pallas_call(
        _argmin_body,
        grid=(BATCH,),
        in_specs=[
            pl.BlockSpec((1, TILE, DIM), lambda b: (b, 0, 0)),
            pl.BlockSpec((N_EMB, DIM), lambda b: (0, 0)),
        ],
        out_specs=[
            pl.BlockSpec((8, SEQ), lambda b: (b // 8, 0)),
            pl.BlockSpec(memory_space=pltpu.SMEM),
        ],
        out_shape=[
            jax.ShapeDtypeStruct((BATCH, SEQ), jnp.int32),
            jax.ShapeDtypeStruct((1, 1), jnp.float32),
        ],
    )(z, codebook)


@functools.cache
def _make_sc_gather():
    mesh = plsc.VectorSubcoreMesh(core_axis_name="c", subcore_axis_name="s")

    @functools.partial(
        pl.kernel,
        mesh=mesh,
        out_type=jax.ShapeDtypeStruct((BATCH, SEQ, DIM), jnp.float32),
        scratch_types=[
            pltpu.VMEM((NGCH, GCH), jnp.int32),
            pltpu.VMEM((BPW, DIM), jnp.float32),
            pltpu.SemaphoreType.DMA,
        ],
        compiler_params=pltpu.CompilerParams(use_tc_tiling_on_sc=False),
    )
    def _sc_gather(cb_hbm, idx_hbm, out_hbm, idx_v, rows_v, sem):
        wid = lax.axis_index("s") * NCORES + lax.axis_index("c")
        b = wid // 2
        half = wid % 2
        pltpu.sync_copy(idx_hbm.at[wid], idx_v)
        copies = [
            pltpu.async_copy(cb_hbm.at[idx_v.at[j]],
                             rows_v.at[pl.ds(j * GCH, GCH)], sem)
            for j in range(NGCH)
        ]
        for c in copies:
            c.wait()
        pltpu.sync_copy(rows_v, out_hbm.at[b, pl.ds(half * BPW, BPW)])

    return _sc_gather


def kernel(z, codebook):
    idx, lsum = _tc_argmin(z, codebook)
    quantized = _make_sc_gather()(
        codebook, idx.reshape(NWORKERS, NGCH, GCH))
    loss = lsum[0, 0]
    return quantized, loss, idx


# speedup vs baseline: 1.4673x; 1.0229x over previous
import functools

import jax
import jax.numpy as jnp
from jax import lax
from jax.experimental import pallas as pl
from jax.experimental.pallas import tpu as pltpu
from jax.experimental.pallas import tpu_sc as plsc

N_EMB = 8192
DIM = 64
BATCH = 16
SEQ = 576
TOKENS = BATCH * SEQ
TILE = 1152
BPS = TILE // SEQ
NSTEPS = TOKENS // TILE
RB = 72
NRB = TILE // RB
NLT = N_EMB // 128

NCORES = 2
NSUB = 16
NWORKERS = NCORES * NSUB
BPW = TOKENS // NWORKERS
GCH = 96
NGCH = BPW // GCH


def _argmin_body(z_ref, cb_ref, idx_ref, loss_ref):
    step = pl.program_id(0)
    lane = lax.broadcasted_iota(jnp.int32, (1, 128), 1).astype(jnp.float32)

    total = jnp.float32(0.0)
    rows = []
    idx_parts = []
    for bb in range(BPS):
      zt = z_ref[bb]
      sz = jnp.sum(zt * zt, axis=1, keepdims=True)
      zneg = zt * -2.0
      for rb in range(SEQ // RB):
        r0 = rb * RB
        zr = zneg[r0:r0 + RB]
        dneg = lax.dot_general(zr, cb_ref[...], (((1,), (1,)), ((), ())),
                               preferred_element_type=jnp.float32)
        szb = jnp.broadcast_to(sz[r0:r0 + RB], (RB, 128))
        bv = szb + dneg[:, 0:128]
        bi = jnp.broadcast_to(lane, (RB, 128))
        for t in range(1, NLT):
            dt = szb + dneg[:, t * 128:(t + 1) * 128]
            ids = lane + jnp.float32(t * 128)
            take = dt < bv
            bv = jnp.where(take, dt, bv)
            bi = jnp.where(take, ids, bi)
        mv = jnp.min(bv, axis=1, keepdims=True)
        mi = jnp.min(jnp.where(bv == mv, bi, jnp.float32(N_EMB)),
                     axis=1, keepdims=True)
        idx_parts.append(mi[:, 0].astype(jnp.int32))
        total = total + jnp.sum(mv[:, 0])
      rows.append(jnp.concatenate(idx_parts).reshape(1, SEQ))
      idx_parts = []

    base = BPS * (step % (8 // BPS))
    for bb in range(BPS):
        idx_ref[pl.ds(base + bb, 1), :] = rows[bb]

    @pl.when(step == 0)
    def _():
        loss_ref[0, 0] = 0.0

    loss_ref[0, 0] += total

    @pl.when(step == NSTEPS - 1)
    def _():
        m = loss_ref[0, 0] / (TOKENS * DIM)
        loss_ref[0, 0] = m + 0.25 * m


def _tc_argmin(z, codebook):
    return pl.pallas_call(
        _argmin_body,
        grid=(NSTEPS,),
        in_specs=[
            pl.BlockSpec((BPS, SEQ, DIM), lambda b: (b, 0, 0)),
            pl.BlockSpec((N_EMB, DIM), lambda b: (0, 0)),
        ],
        out_specs=[
            pl.BlockSpec((8, SEQ), lambda b: (b // (8 // BPS), 0)),
            pl.BlockSpec(memory_space=pltpu.SMEM),
        ],
        out_shape=[
            jax.ShapeDtypeStruct((BATCH, SEQ), jnp.int32),
            jax.ShapeDtypeStruct((1, 1), jnp.float32),
        ],
    )(z, codebook)


@functools.cache
def _make_sc_gather():
    mesh = plsc.VectorSubcoreMesh(core_axis_name="c", subcore_axis_name="s")

    @functools.partial(
        pl.kernel,
        mesh=mesh,
        out_type=jax.ShapeDtypeStruct((BATCH, SEQ, DIM), jnp.float32),
        scratch_types=[
            pltpu.VMEM((NGCH, GCH), jnp.int32),
            pltpu.VMEM((BPW, DIM), jnp.float32),
            pltpu.SemaphoreType.DMA,
        ],
        compiler_params=pltpu.CompilerParams(use_tc_tiling_on_sc=False),
    )
    def _sc_gather(cb_hbm, idx_hbm, out_hbm, idx_v, rows_v, sem):
        wid = lax.axis_index("s") * NCORES + lax.axis_index("c")
        b = wid // 2
        half = wid % 2
        pltpu.sync_copy(idx_hbm.at[wid], idx_v)
        copies = [
            pltpu.async_copy(cb_hbm.at[idx_v.at[j]],
                             rows_v.at[pl.ds(j * GCH, GCH)], sem)
            for j in range(NGCH)
        ]
        for c in copies:
            c.wait()
        pltpu.sync_copy(rows_v, out_hbm.at[b, pl.ds(half * BPW, BPW)])

    return _sc_gather


def kernel(z, codebook):
    idx, lsum = _tc_argmin(z, codebook)
    quantized = _make_sc_gather()(
        codebook, idx.reshape(NWORKERS, NGCH, GCH))
    loss = lsum[0, 0]
    return quantized, loss, idx


# speedup vs baseline: 1.8289x; 1.2464x over previous
import functools

import jax
import jax.numpy as jnp
from jax import lax
from jax.experimental import pallas as pl
from jax.experimental.pallas import tpu as pltpu
from jax.experimental.pallas import tpu_sc as plsc

N_EMB = 8192
DIM = 64
BATCH = 16
SEQ = 576
TOKENS = BATCH * SEQ
TILE = 1152
BPS = TILE // SEQ
NSTEPS = TOKENS // TILE
RB = 96
NRB = TILE // RB
NLT = N_EMB // 128

NCORES = 2
NSUB = 16
NWORKERS = NCORES * NSUB
BPW = TOKENS // NWORKERS
GCH = 96
NGCH = BPW // GCH


def _argmin_body(z_ref, cb_ref, idx_ref, loss_ref):
    step = pl.program_id(0)
    lane = lax.broadcasted_iota(jnp.int32, (1, 128), 1).astype(jnp.float32)

    total = jnp.float32(0.0)
    rows = []
    idx_parts = []
    for bb in range(BPS):
      zt = z_ref[bb]
      sz = jnp.sum(zt * zt, axis=1, keepdims=True)
      zneg = zt * -2.0
      for rb in range(SEQ // RB):
        r0 = rb * RB
        zr = zneg[r0:r0 + RB]
        dneg = lax.dot_general(zr, cb_ref[...], (((1,), (1,)), ((), ())),
                               preferred_element_type=jnp.float32)
        szb = jnp.broadcast_to(sz[r0:r0 + RB], (RB, 128))
        bv = szb + dneg[:, 0:128]
        bi = jnp.broadcast_to(lane, (RB, 128))
        for t in range(1, NLT):
            dt = szb + dneg[:, t * 128:(t + 1) * 128]
            ids = lane + jnp.float32(t * 128)
            take = dt < bv
            bv = jnp.where(take, dt, bv)
            bi = jnp.where(take, ids, bi)
        mv = jnp.min(bv, axis=1, keepdims=True)
        mi = jnp.min(jnp.where(bv == mv, bi, jnp.float32(N_EMB)),
                     axis=1, keepdims=True)
        idx_parts.append(mi[:, 0].astype(jnp.int32))
        total = total + jnp.sum(mv[:, 0])
      rows.append(jnp.concatenate(idx_parts).reshape(1, SEQ))
      idx_parts = []

    base = BPS * (step % (8 // BPS))
    for bb in range(BPS):
        idx_ref[pl.ds(base + bb, 1), :] = rows[bb]

    @pl.when(step == 0)
    def _():
        loss_ref[0, 0] = 0.0

    loss_ref[0, 0] += total

    @pl.when(step == NSTEPS - 1)
    def _():
        m = loss_ref[0, 0] / (TOKENS * DIM)
        loss_ref[0, 0] = m + 0.25 * m


def _tc_argmin(z, codebook):
    return pl.pallas_call(
        _argmin_body,
        grid=(NSTEPS,),
        in_specs=[
            pl.BlockSpec((BPS, SEQ, DIM), lambda b: (b, 0, 0)),
            pl.BlockSpec((N_EMB, DIM), lambda b: (0, 0)),
        ],
        out_specs=[
            pl.BlockSpec((8, SEQ), lambda b: (b // (8 // BPS), 0)),
            pl.BlockSpec(memory_space=pltpu.SMEM),
        ],
        out_shape=[
            jax.ShapeDtypeStruct((BATCH, SEQ), jnp.int32),
            jax.ShapeDtypeStruct((1, 1), jnp.float32),
        ],
    )(z, codebook)


@functools.cache
def _make_sc_gather():
    mesh = plsc.VectorSubcoreMesh(core_axis_name="c", subcore_axis_name="s")

    @functools.partial(
        pl.kernel,
        mesh=mesh,
        out_type=jax.ShapeDtypeStruct((BATCH, SEQ, DIM), jnp.float32),
        scratch_types=[
            pltpu.VMEM((NGCH, GCH), jnp.int32),
            pltpu.VMEM((BPW, DIM), jnp.float32),
            pltpu.SemaphoreType.DMA,
        ],
        compiler_params=pltpu.CompilerParams(use_tc_tiling_on_sc=False),
    )
    def _sc_gather(cb_hbm, idx_hbm, out_hbm, idx_v, rows_v, sem):
        wid = lax.axis_index("s") * NCORES + lax.axis_index("c")
        b = wid // 2
        half = wid % 2
        pltpu.sync_copy(idx_hbm.at[wid], idx_v)
        copies = [
            pltpu.async_copy(cb_hbm.at[idx_v.at[j]],
                             rows_v.at[pl.ds(j * GCH, GCH)], sem)
            for j in range(NGCH)
        ]
        for c in copies:
            c.wait()
        pltpu.sync_copy(rows_v, out_hbm.at[b, pl.ds(half * BPW, BPW)])

    return _sc_gather


def kernel(z, codebook):
    idx, lsum = _tc_argmin(z, codebook)
    quantized = _make_sc_gather()(
        codebook, idx.reshape(NWORKERS, NGCH, GCH))
    loss = lsum[0, 0]
    return quantized, loss, idx


# speedup vs baseline: 2.0690x; 1.1312x over previous
import functools

import jax
import jax.numpy as jnp
from jax import lax
from jax.experimental import pallas as pl
from jax.experimental.pallas import tpu as pltpu
from jax.experimental.pallas import tpu_sc as plsc

N_EMB = 8192
DIM = 64
BATCH = 16
SEQ = 576
TOKENS = BATCH * SEQ
TILE = 1152
BPS = TILE // SEQ
NSTEPS = TOKENS // TILE
RB = 144
NRB = TILE // RB
NLT = N_EMB // 128

NCORES = 2
NSUB = 16
NWORKERS = NCORES * NSUB
BPW = TOKENS // NWORKERS
GCH = 96
NGCH = BPW // GCH


def _argmin_body(z_ref, cb_ref, idx_ref, loss_ref):
    step = pl.program_id(0)
    lane = lax.broadcasted_iota(jnp.int32, (1, 128), 1).astype(jnp.float32)

    total = jnp.float32(0.0)
    rows = []
    idx_parts = []
    for bb in range(BPS):
      zt = z_ref[bb]
      sz = jnp.sum(zt * zt, axis=1, keepdims=True)
      zneg = zt * -2.0
      for rb in range(SEQ // RB):
        r0 = rb * RB
        zr = zneg[r0:r0 + RB]
        dneg = lax.dot_general(zr, cb_ref[...], (((1,), (1,)), ((), ())),
                               preferred_element_type=jnp.float32)
        szb = jnp.broadcast_to(sz[r0:r0 + RB], (RB, 128))
        bv = szb + dneg[:, 0:128]
        bi = jnp.broadcast_to(lane, (RB, 128))
        for t in range(1, NLT):
            dt = szb + dneg[:, t * 128:(t + 1) * 128]
            ids = lane + jnp.float32(t * 128)
            take = dt < bv
            bv = jnp.where(take, dt, bv)
            bi = jnp.where(take, ids, bi)
        mv = jnp.min(bv, axis=1, keepdims=True)
        mi = jnp.min(jnp.where(bv == mv, bi, jnp.float32(N_EMB)),
                     axis=1, keepdims=True)
        idx_parts.append(mi[:, 0].astype(jnp.int32))
        total = total + jnp.sum(mv[:, 0])
      rows.append(jnp.concatenate(idx_parts).reshape(1, SEQ))
      idx_parts = []

    base = BPS * (step % (8 // BPS))
    for bb in range(BPS):
        idx_ref[pl.ds(base + bb, 1), :] = rows[bb]

    @pl.when(step == 0)
    def _():
        loss_ref[0, 0] = 0.0

    loss_ref[0, 0] += total

    @pl.when(step == NSTEPS - 1)
    def _():
        m = loss_ref[0, 0] / (TOKENS * DIM)
        loss_ref[0, 0] = m + 0.25 * m


def _tc_argmin(z, codebook):
    return pl.pallas_call(
        _argmin_body,
        grid=(NSTEPS,),
        in_specs=[
            pl.BlockSpec((BPS, SEQ, DIM), lambda b: (b, 0, 0)),
            pl.BlockSpec((N_EMB, DIM), lambda b: (0, 0)),
        ],
        out_specs=[
            pl.BlockSpec((8, SEQ), lambda b: (b // (8 // BPS), 0)),
            pl.BlockSpec(memory_space=pltpu.SMEM),
        ],
        out_shape=[
            jax.ShapeDtypeStruct((BATCH, SEQ), jnp.int32),
            jax.ShapeDtypeStruct((1, 1), jnp.float32),
        ],
    )(z, codebook)


@functools.cache
def _make_sc_gather():
    mesh = plsc.VectorSubcoreMesh(core_axis_name="c", subcore_axis_name="s")

    @functools.partial(
        pl.kernel,
        mesh=mesh,
        out_type=jax.ShapeDtypeStruct((BATCH, SEQ, DIM), jnp.float32),
        scratch_types=[
            pltpu.VMEM((NGCH, GCH), jnp.int32),
            pltpu.VMEM((BPW, DIM), jnp.float32),
            pltpu.SemaphoreType.DMA,
        ],
        compiler_params=pltpu.CompilerParams(use_tc_tiling_on_sc=False),
    )
    def _sc_gather(cb_hbm, idx_hbm, out_hbm, idx_v, rows_v, sem):
        wid = lax.axis_index("s") * NCORES + lax.axis_index("c")
        b = wid // 2
        half = wid % 2
        pltpu.sync_copy(idx_hbm.at[wid], idx_v)
        copies = [
            pltpu.async_copy(cb_hbm.at[idx_v.at[j]],
                             rows_v.at[pl.ds(j * GCH, GCH)], sem)
            for j in range(NGCH)
        ]
        for c in copies:
            c.wait()
        pltpu.sync_copy(rows_v, out_hbm.at[b, pl.ds(half * BPW, BPW)])

    return _sc_gather


def kernel(z, codebook):
    idx, lsum = _tc_argmin(z, codebook)
    quantized = _make_sc_gather()(
        codebook, idx.reshape(NWORKERS, NGCH, GCH))
    loss = lsum[0, 0]
    return quantized, loss, idx
